# Initial kernel scaffold; baseline (speedup 1.0000x reference)
import jax
import jax.numpy as jnp
from jax.experimental import pallas as pl


def kernel(x, edge_index, params):
    raise NotImplementedError("write your pallas kernel here")


# speedup vs baseline: 2.0381x; 2.0381x over previous
import functools

import jax
import jax.numpy as jnp
from jax import lax
from jax.experimental import pallas as pl
from jax.experimental.pallas import tpu as pltpu
from jax.experimental.pallas import tpu_sc as plsc

F32 = jnp.float32
D = 128
SRCW = 272
DSTW = 144
AGGW = 144
NC, NS = 2, 16
K = 16
TC_ROWS = 1000


def _mm_t(x, w):
    return lax.dot_general(x, w, (((1,), (1,)), ((), ())),
                           preferred_element_type=F32)


def _build_tables(x, p, src_ref, dst_ref):
    h = _mm_t(x, p["W"][...]) + p["Wb"][...]
    hU = _mm_t(h, p["U"][...]) + p["Ub"][...]
    hV = _mm_t(h, p["V"][...]) + p["Vb"][...]
    a_s = jnp.dot(h, p["A2p"][...], preferred_element_type=F32)
    a_d = jnp.dot(h, p["A1p"][...], preferred_element_type=F32) + p["Abp"][...]
    src_ref[...] = jnp.concatenate([h, hV, a_s], axis=1)
    dst_ref[...] = jnp.concatenate([hU, a_d], axis=1)


def _tables_body(x_ref, W, Wb, U, Ub, V, Vb, A1p, A2p, Abp, src_ref, dst_ref):
    p = {"W": W, "Wb": Wb, "U": U, "Ub": Ub, "V": V, "Vb": Vb,
         "A1p": A1p, "A2p": A2p, "Abp": Abp}
    _build_tables(x_ref[...], p, src_ref, dst_ref)


def _combine(p_ref, g, b):
    psum = p_ref[0] + p_ref[1]
    s = psum[:, 128:129]
    aggr = psum[:, :D] / (s + 1e-16)
    mu = jnp.mean(aggr, axis=-1, keepdims=True)
    var = jnp.mean((aggr - mu) ** 2, axis=-1, keepdims=True)
    return (aggr - mu) * lax.rsqrt(var + 1e-5) * g[...] + b[...]


def _combine_tables_body(p_ref, x_ref, g0, b0,
                         W, Wb, U, Ub, V, Vb, A1p, A2p, Abp,
                         src_ref, dst_ref):
    y = _combine(p_ref, g0, b0)
    x1 = jnp.maximum(y + x_ref[...], 0.0)
    p = {"W": W, "Wb": Wb, "U": U, "Ub": Ub, "V": V, "Vb": Vb,
         "A1p": A1p, "A2p": A2p, "Abp": Abp}
    _build_tables(x1, p, src_ref, dst_ref)


def _combine_final_body(p_ref, g1, b1, out_ref):
    out_ref[...] = _combine(p_ref, g1, b1)


def _row_spec(w):
    return pl.BlockSpec((TC_ROWS, w), lambda i: (i, 0))


def _full_spec(shape):
    nd = len(shape)
    return pl.BlockSpec(shape, lambda i, _n=nd: (0,) * _n)


def _prep_params(p):
    A = p["A"]
    A1 = A[0, :D]
    A2 = A[0, D:]
    A1p = jnp.zeros((D, 16), F32).at[:, 0].set(A1)
    A2p = jnp.zeros((D, 16), F32).at[:, 0].set(A2)
    Abp = jnp.zeros((1, 16), F32).at[0, 0].set(p["Ab"][0])
    return {"W": p["W"], "Wb": p["Wb"].reshape(1, D),
            "U": p["U"], "Ub": p["Ub"].reshape(1, D),
            "V": p["V"], "Vb": p["Vb"].reshape(1, D),
            "A1p": A1p, "A2p": A2p, "Abp": Abp,
            "g": p["ln_g"].reshape(1, D), "b": p["ln_b"].reshape(1, D)}


def _weight_args(q):
    ws = [q["W"], q["Wb"], q["U"], q["Ub"], q["V"], q["Vb"],
          q["A1p"], q["A2p"], q["Abp"]]
    return ws, [_full_spec(w.shape) for w in ws]


def _sc_edge_body(n_nodes, n_edges,
                  stab, dtab, sidx_h, didx_h, out,
                  sidx, didx, srows, drows, msg, sem, aggr):
    ept = n_edges // (NC * NS)
    npad = ((n_nodes + NS * K - 1) // (NS * K)) * (NS * K)
    rpt = npad // NS
    nchunk = ept // K
    c = lax.axis_index("c")
    s = lax.axis_index("s")
    wid = c * NS + s

    zeros16 = jnp.zeros((16,), F32)
    iota = lax.iota(jnp.int32, 16)

    def _zm(j, _):
        for gcol in range(AGGW // 16):
            msg[j, pl.ds(gcol * 16, 16)] = zeros16
        return 0
    lax.fori_loop(0, K, _zm, 0)
    for i in range(rpt // K):
        pltpu.sync_copy(msg, aggr.at[pl.ds(s * rpt + i * K, K)])
    plsc.subcore_barrier()

    col_as = jnp.full((16,), D + D, jnp.int32)
    col_ad = jnp.full((16,), D, jnp.int32)

    def _chunk(ci, _):
        eb = wid * ept + ci * K
        pltpu.sync_copy(sidx_h.at[pl.ds(eb, K)], sidx)
        pltpu.sync_copy(didx_h.at[pl.ds(eb, K)], didx)
        d1 = pltpu.async_copy(stab.at[sidx], srows, sem)
        d2 = pltpu.async_copy(dtab.at[didx], drows, sem)
        d1.wait()
        d2.wait()

        a_s = plsc.load_gather(srows, [iota, col_as])
        a_d = plsc.load_gather(drows, [iota, col_ad])
        logit = a_s + a_d
        ex = jnp.exp(jnp.maximum(logit, logit * 0.2))
        plsc.store_scatter(msg, [iota, col_ad], ex)

        def _edge(j, _):
            jv = jnp.full((16,), j, jnp.int32)
            exv = plsc.load_gather(msg, [jv, col_ad])
            for f in range(D // 16):
                hU = drows[j, pl.ds(f * 16, 16)]
                hV = srows[j, pl.ds(D + f * 16, 16)]
                hj = srows[j, pl.ds(f * 16, 16)]
                gate = 1.0 / (1.0 + jnp.exp(-(hU + hV)))
                msg[j, pl.ds(f * 16, 16)] = gate * exv * hj
            return 0
        lax.fori_loop(0, K, _edge, 0)

        pltpu.sync_copy(msg, aggr.at[didx], add=True)
        return 0
    lax.fori_loop(0, nchunk, _chunk, 0)

    plsc.subcore_barrier()
    full = n_nodes // rpt
    rem = n_nodes - full * rpt

    @pl.when(s < full)
    def _():
        pltpu.sync_copy(aggr.at[pl.ds(s * rpt, rpt)],
                        out.at[c, pl.ds(s * rpt, rpt)])
    if rem:
        @pl.when(s == full)
        def _():
            pltpu.sync_copy(aggr.at[pl.ds(full * rpt, rem)],
                            out.at[c, pl.ds(full * rpt, rem)])


def _sc_edge(src_tab, dst_tab, src_idx, dst_idx):
    n_nodes = src_tab.shape[0]
    n_edges = src_idx.shape[0]
    npad = ((n_nodes + NS * K - 1) // (NS * K)) * (NS * K)
    mesh = plsc.VectorSubcoreMesh(core_axis_name="c", subcore_axis_name="s")
    run = pl.kernel(
        functools.partial(_sc_edge_body, n_nodes, n_edges),
        out_type=jax.ShapeDtypeStruct((NC, n_nodes, AGGW), F32),
        mesh=mesh,
        compiler_params=pltpu.CompilerParams(use_tc_tiling_on_sc=False,
                                             needs_layout_passes=False),
        scratch_types=[
            pltpu.VMEM((K,), jnp.int32),
            pltpu.VMEM((K,), jnp.int32),
            pltpu.VMEM((K, SRCW), F32),
            pltpu.VMEM((K, DSTW), F32),
            pltpu.VMEM((K, AGGW), F32),
            pltpu.SemaphoreType.DMA,
            pltpu.VMEM_SHARED((npad, AGGW), F32),
        ],
    )
    return run(src_tab, dst_tab, src_idx, dst_idx)


def kernel(x, edge_index, params):
    n = x.shape[0]
    grid = (n // TC_ROWS,)
    src_idx = edge_index[0]
    dst_idx = edge_index[1]
    q0 = _prep_params(params["l0"])
    q1 = _prep_params(params["l1"])

    tab_shapes = [jax.ShapeDtypeStruct((n, SRCW), F32),
                  jax.ShapeDtypeStruct((n, DSTW), F32)]
    tab_specs = [_row_spec(SRCW), _row_spec(DSTW)]

    w0, w0_specs = _weight_args(q0)
    stab0, dtab0 = pl.pallas_call(
        _tables_body,
        grid=grid,
        in_specs=[_row_spec(D)] + w0_specs,
        out_specs=tab_specs,
        out_shape=tab_shapes,
    )(x, *w0)

    part0 = _sc_edge(stab0, dtab0, src_idx, dst_idx)

    w1, w1_specs = _weight_args(q1)
    stab1, dtab1 = pl.pallas_call(
        _combine_tables_body,
        grid=grid,
        in_specs=[pl.BlockSpec((NC, TC_ROWS, AGGW), lambda i: (0, i, 0)),
                  _row_spec(D), _full_spec((1, D)), _full_spec((1, D))]
                 + w1_specs,
        out_specs=tab_specs,
        out_shape=tab_shapes,
    )(part0, x, q0["g"], q0["b"], *w1)

    part1 = _sc_edge(stab1, dtab1, src_idx, dst_idx)

    out = pl.pallas_call(
        _combine_final_body,
        grid=grid,
        in_specs=[pl.BlockSpec((NC, TC_ROWS, AGGW), lambda i: (0, i, 0)),
                  _full_spec((1, D)), _full_spec((1, D))],
        out_specs=_row_spec(D),
        out_shape=jax.ShapeDtypeStruct((n, D), F32),
    )(part1, q1["g"], q1["b"])
    return out


# speedup vs baseline: 3.2570x; 1.5980x over previous
import functools

import jax
import jax.numpy as jnp
from jax import lax
from jax.experimental import pallas as pl
from jax.experimental.pallas import tpu as pltpu
from jax.experimental.pallas import tpu_sc as plsc

F32 = jnp.float32
D = 128
SRCW = 272
DSTW = 144
AGGW = 144
NC, NS = 2, 16
K = 16
TC_ROWS = 1000


def _mm_t(x, w):
    return lax.dot_general(x, w, (((1,), (1,)), ((), ())),
                           preferred_element_type=F32)


def _build_tables(x, p, src_ref, dst_ref):
    h = _mm_t(x, p["W"][...]) + p["Wb"][...]
    hU = _mm_t(h, p["U"][...]) + p["Ub"][...]
    hV = _mm_t(h, p["V"][...]) + p["Vb"][...]
    a_s = jnp.dot(h, p["A2p"][...], preferred_element_type=F32)
    a_d = jnp.dot(h, p["A1p"][...], preferred_element_type=F32) + p["Abp"][...]
    src_ref[...] = jnp.concatenate([h, hV, a_s], axis=1)
    dst_ref[...] = jnp.concatenate([hU, a_d], axis=1)


def _tables_body(x_ref, W, Wb, U, Ub, V, Vb, A1p, A2p, Abp, src_ref, dst_ref):
    p = {"W": W, "Wb": Wb, "U": U, "Ub": Ub, "V": V, "Vb": Vb,
         "A1p": A1p, "A2p": A2p, "Abp": Abp}
    _build_tables(x_ref[...], p, src_ref, dst_ref)


def _combine(p_ref, g, b):
    psum = p_ref[0] + p_ref[1]
    s = psum[:, 128:129]
    aggr = psum[:, :D] / (s + 1e-16)
    mu = jnp.mean(aggr, axis=-1, keepdims=True)
    var = jnp.mean((aggr - mu) ** 2, axis=-1, keepdims=True)
    return (aggr - mu) * lax.rsqrt(var + 1e-5) * g[...] + b[...]


def _combine_tables_body(p_ref, x_ref, g0, b0,
                         W, Wb, U, Ub, V, Vb, A1p, A2p, Abp,
                         src_ref, dst_ref):
    y = _combine(p_ref, g0, b0)
    x1 = jnp.maximum(y + x_ref[...], 0.0)
    p = {"W": W, "Wb": Wb, "U": U, "Ub": Ub, "V": V, "Vb": Vb,
         "A1p": A1p, "A2p": A2p, "Abp": Abp}
    _build_tables(x1, p, src_ref, dst_ref)


def _combine_final_body(p_ref, g1, b1, out_ref):
    out_ref[...] = _combine(p_ref, g1, b1)


def _row_spec(w):
    return pl.BlockSpec((TC_ROWS, w), lambda i: (i, 0))


def _full_spec(shape):
    nd = len(shape)
    return pl.BlockSpec(shape, lambda i, _n=nd: (0,) * _n)


def _prep_params(p):
    A = p["A"]
    A1 = A[0, :D]
    A2 = A[0, D:]
    A1p = jnp.zeros((D, 16), F32).at[:, 0].set(A1)
    A2p = jnp.zeros((D, 16), F32).at[:, 0].set(A2)
    Abp = jnp.zeros((1, 16), F32).at[0, 0].set(p["Ab"][0])
    return {"W": p["W"], "Wb": p["Wb"].reshape(1, D),
            "U": p["U"], "Ub": p["Ub"].reshape(1, D),
            "V": p["V"], "Vb": p["Vb"].reshape(1, D),
            "A1p": A1p, "A2p": A2p, "Abp": Abp,
            "g": p["ln_g"].reshape(1, D), "b": p["ln_b"].reshape(1, D)}


def _weight_args(q):
    ws = [q["W"], q["Wb"], q["U"], q["Ub"], q["V"], q["Vb"],
          q["A1p"], q["A2p"], q["Abp"]]
    return ws, [_full_spec(w.shape) for w in ws]


def _sc_edge_body(n_nodes, n_edges,
                  stab, dtab, sidx_h, didx_h, out,
                  sidx, didx, srows, drows, msg, semg, sems, aggr):
    ept = n_edges // (NC * NS)
    npad = ((n_nodes + NS * K - 1) // (NS * K)) * (NS * K)
    rpt = npad // NS
    nchunk = ept // K
    c = lax.axis_index("c")
    s = lax.axis_index("s")
    wid = c * NS + s

    zeros16 = jnp.zeros((16,), F32)
    iota = lax.iota(jnp.int32, 16)

    for m in range(2):
        def _zm(j, _, _m=m):
            for gcol in range(AGGW // 16):
                msg[_m, j, pl.ds(gcol * 16, 16)] = zeros16
            return 0
        lax.fori_loop(0, K, _zm, 0)
    for i in range(rpt // K):
        pltpu.sync_copy(msg.at[0], aggr.at[pl.ds(s * rpt + i * K, K)])
    plsc.subcore_barrier()

    pltpu.sync_copy(sidx_h.at[pl.ds(wid * nchunk, nchunk)], sidx)
    pltpu.sync_copy(didx_h.at[pl.ds(wid * nchunk, nchunk)], didx)

    col_as = jnp.full((16,), D + D, jnp.int32)
    col_ad = jnp.full((16,), D, jnp.int32)

    def _issue(ci, b):
        pltpu.async_copy(stab.at[sidx.at[ci]], srows.at[b], semg.at[b])
        pltpu.async_copy(dtab.at[didx.at[ci]], drows.at[b], semg.at[b])

    def _wait_gather(ci, b):
        pltpu.make_async_copy(stab.at[sidx.at[ci]], srows.at[b],
                              semg.at[b]).wait()
        pltpu.make_async_copy(dtab.at[didx.at[ci]], drows.at[b],
                              semg.at[b]).wait()

    def _compute(ci, b):
        sr = srows.at[b]
        dr = drows.at[b]
        mg = msg.at[b]
        a_s = plsc.load_gather(sr, [iota, col_as])
        a_d = plsc.load_gather(dr, [iota, col_ad])
        logit = a_s + a_d
        ex = jnp.exp(jnp.maximum(logit, logit * 0.2))
        plsc.store_scatter(mg, [iota, col_ad], ex)

        def _edge(j, _):
            jv = jnp.full((16,), j, jnp.int32)
            exv = plsc.load_gather(mg, [jv, col_ad])
            for f in range(D // 16):
                hU = dr[j, pl.ds(f * 16, 16)]
                hV = sr[j, pl.ds(D + f * 16, 16)]
                hj = sr[j, pl.ds(f * 16, 16)]
                gate = 1.0 / (1.0 + jnp.exp(-(hU + hV)))
                mg[j, pl.ds(f * 16, 16)] = gate * exv * hj
            return 0
        lax.fori_loop(0, K, _edge, 0)
        pltpu.async_copy(msg.at[b], aggr.at[didx.at[ci]], sems.at[b],
                         add=True)

    def _wait_scatter(ci, b):
        pltpu.make_async_copy(msg.at[b], aggr.at[didx.at[ci]],
                              sems.at[b]).wait()

    _issue(0, 0)

    def _pair(m, _):
        c0 = 2 * m
        c1 = 2 * m + 1
        _wait_gather(c0, 0)
        _issue(c1, 1)

        @pl.when(m >= 1)
        def _():
            _wait_scatter(c0 - 2, 0)
        _compute(c0, 0)
        _wait_gather(c1, 1)
        if nchunk % 2:
            _issue(c1 + 1, 0)
        else:
            @pl.when(c1 + 1 < nchunk)
            def _():
                _issue(c1 + 1, 0)

        @pl.when(m >= 1)
        def _():
            _wait_scatter(c1 - 2, 1)
        _compute(c1, 1)
        return 0
    npairs = nchunk // 2
    lax.fori_loop(0, npairs, _pair, 0)

    if nchunk % 2:
        last = nchunk - 1
        _wait_gather(last, 0)
        _wait_scatter(last - 2, 0)
        _compute(last, 0)
        _wait_scatter(last - 1, 1)
        _wait_scatter(last, 0)
    else:
        _wait_scatter(nchunk - 2, 0)
        _wait_scatter(nchunk - 1, 1)

    plsc.subcore_barrier()
    full = n_nodes // rpt
    rem = n_nodes - full * rpt

    @pl.when(s < full)
    def _():
        pltpu.sync_copy(aggr.at[pl.ds(s * rpt, rpt)],
                        out.at[c, pl.ds(s * rpt, rpt)])
    if rem:
        @pl.when(s == full)
        def _():
            pltpu.sync_copy(aggr.at[pl.ds(full * rpt, rem)],
                            out.at[c, pl.ds(full * rpt, rem)])


def _sc_edge(src_tab, dst_tab, src_idx, dst_idx):
    n_nodes = src_tab.shape[0]
    n_edges = src_idx.shape[0]
    npad = ((n_nodes + NS * K - 1) // (NS * K)) * (NS * K)
    nchunk = n_edges // (NC * NS * K)
    mesh = plsc.VectorSubcoreMesh(core_axis_name="c", subcore_axis_name="s")
    run = pl.kernel(
        functools.partial(_sc_edge_body, n_nodes, n_edges),
        out_type=jax.ShapeDtypeStruct((NC, n_nodes, AGGW), F32),
        mesh=mesh,
        compiler_params=pltpu.CompilerParams(use_tc_tiling_on_sc=False,
                                             needs_layout_passes=False),
        scratch_types=[
            pltpu.VMEM((nchunk, 16), jnp.int32),
            pltpu.VMEM((nchunk, 16), jnp.int32),
            pltpu.VMEM((2, K, SRCW), F32),
            pltpu.VMEM((2, K, DSTW), F32),
            pltpu.VMEM((2, K, AGGW), F32),
            pltpu.SemaphoreType.DMA((2,)),
            pltpu.SemaphoreType.DMA((2,)),
            pltpu.VMEM_SHARED((npad, AGGW), F32),
        ],
    )
    return run(src_tab, dst_tab,
               src_idx.reshape(-1, 16), dst_idx.reshape(-1, 16))


def kernel(x, edge_index, params):
    n = x.shape[0]
    grid = (n // TC_ROWS,)
    src_idx = edge_index[0]
    dst_idx = edge_index[1]
    q0 = _prep_params(params["l0"])
    q1 = _prep_params(params["l1"])

    tab_shapes = [jax.ShapeDtypeStruct((n, SRCW), F32),
                  jax.ShapeDtypeStruct((n, DSTW), F32)]
    tab_specs = [_row_spec(SRCW), _row_spec(DSTW)]

    w0, w0_specs = _weight_args(q0)
    stab0, dtab0 = pl.pallas_call(
        _tables_body,
        grid=grid,
        in_specs=[_row_spec(D)] + w0_specs,
        out_specs=tab_specs,
        out_shape=tab_shapes,
    )(x, *w0)

    part0 = _sc_edge(stab0, dtab0, src_idx, dst_idx)

    w1, w1_specs = _weight_args(q1)
    stab1, dtab1 = pl.pallas_call(
        _combine_tables_body,
        grid=grid,
        in_specs=[pl.BlockSpec((NC, TC_ROWS, AGGW), lambda i: (0, i, 0)),
                  _row_spec(D), _full_spec((1, D)), _full_spec((1, D))]
                 + w1_specs,
        out_specs=tab_specs,
        out_shape=tab_shapes,
    )(part0, x, q0["g"], q0["b"], *w1)

    part1 = _sc_edge(stab1, dtab1, src_idx, dst_idx)

    out = pl.pallas_call(
        _combine_final_body,
        grid=grid,
        in_specs=[pl.BlockSpec((NC, TC_ROWS, AGGW), lambda i: (0, i, 0)),
                  _full_spec((1, D)), _full_spec((1, D))],
        out_specs=_row_spec(D),
        out_shape=jax.ShapeDtypeStruct((n, D), F32),
    )(part1, q1["g"], q1["b"])
    return out


# speedup vs baseline: 8.9940x; 2.7614x over previous
import functools

import jax
import jax.numpy as jnp
from jax import lax
from jax.experimental import pallas as pl
from jax.experimental.pallas import tpu as pltpu
from jax.experimental.pallas import tpu_sc as plsc

F32 = jnp.float32
D = 128
SRCW = 272
DSTW = 144
AGGW = 144
NC, NS = 2, 16
K = 16
TC_ROWS = 1000


def _mm_t(x, w):
    return lax.dot_general(x, w, (((1,), (1,)), ((), ())),
                           preferred_element_type=F32)


def _build_tables(x, p, src_ref, dst_ref):
    h = _mm_t(x, p["W"][...]) + p["Wb"][...]
    hU = _mm_t(h, p["U"][...]) + p["Ub"][...]
    hV = _mm_t(h, p["V"][...]) + p["Vb"][...]
    a_s = jnp.dot(h, p["A2p"][...], preferred_element_type=F32)
    a_d = jnp.dot(h, p["A1p"][...], preferred_element_type=F32) + p["Abp"][...]
    src_ref[...] = jnp.concatenate([h, hV, a_s], axis=1)
    dst_ref[...] = jnp.concatenate([hU, a_d], axis=1)


def _tables_body(x_ref, W, Wb, U, Ub, V, Vb, A1p, A2p, Abp, src_ref, dst_ref):
    p = {"W": W, "Wb": Wb, "U": U, "Ub": Ub, "V": V, "Vb": Vb,
         "A1p": A1p, "A2p": A2p, "Abp": Abp}
    _build_tables(x_ref[...], p, src_ref, dst_ref)


def _combine(p_ref, g, b):
    psum = p_ref[0] + p_ref[1]
    s = psum[:, 128:129]
    aggr = psum[:, :D] / (s + 1e-16)
    mu = jnp.mean(aggr, axis=-1, keepdims=True)
    var = jnp.mean((aggr - mu) ** 2, axis=-1, keepdims=True)
    return (aggr - mu) * lax.rsqrt(var + 1e-5) * g[...] + b[...]


def _combine_tables_body(p_ref, x_ref, g0, b0,
                         W, Wb, U, Ub, V, Vb, A1p, A2p, Abp,
                         src_ref, dst_ref):
    y = _combine(p_ref, g0, b0)
    x1 = jnp.maximum(y + x_ref[...], 0.0)
    p = {"W": W, "Wb": Wb, "U": U, "Ub": Ub, "V": V, "Vb": Vb,
         "A1p": A1p, "A2p": A2p, "Abp": Abp}
    _build_tables(x1, p, src_ref, dst_ref)


def _combine_final_body(p_ref, g1, b1, out_ref):
    out_ref[...] = _combine(p_ref, g1, b1)


def _row_spec(w):
    return pl.BlockSpec((TC_ROWS, w), lambda i: (i, 0))


def _full_spec(shape):
    nd = len(shape)
    return pl.BlockSpec(shape, lambda i, _n=nd: (0,) * _n)


def _prep_params(p):
    A = p["A"]
    A1 = A[0, :D]
    A2 = A[0, D:]
    A1p = jnp.zeros((D, 16), F32).at[:, 0].set(A1)
    A2p = jnp.zeros((D, 16), F32).at[:, 0].set(A2)
    Abp = jnp.zeros((1, 16), F32).at[0, 0].set(p["Ab"][0])
    return {"W": p["W"], "Wb": p["Wb"].reshape(1, D),
            "U": p["U"], "Ub": p["Ub"].reshape(1, D),
            "V": p["V"], "Vb": p["Vb"].reshape(1, D),
            "A1p": A1p, "A2p": A2p, "Abp": Abp,
            "g": p["ln_g"].reshape(1, D), "b": p["ln_b"].reshape(1, D)}


def _weight_args(q):
    ws = [q["W"], q["Wb"], q["U"], q["Ub"], q["V"], q["Vb"],
          q["A1p"], q["A2p"], q["Abp"]]
    return ws, [_full_spec(w.shape) for w in ws]


def _sc_edge_body(n_nodes, n_edges,
                  stab, dtab, sidx_h, didx_h, out,
                  sidx, didx, srows, drows, msg, semg, sems, aggr):
    ept = n_edges // (NC * NS)
    npad = ((n_nodes + NS * K - 1) // (NS * K)) * (NS * K)
    rpt = npad // NS
    nchunk = ept // K
    c = lax.axis_index("c")
    s = lax.axis_index("s")
    wid = c * NS + s

    zeros16 = jnp.zeros((16,), F32)
    iota = lax.iota(jnp.int32, 16)

    for m in range(2):
        def _zm(j, _, _m=m):
            for gcol in range(AGGW // 16):
                msg[_m, j, pl.ds(gcol * 16, 16)] = zeros16
            return 0
        lax.fori_loop(0, K, _zm, 0)
    for i in range(rpt // K):
        pltpu.sync_copy(msg.at[0], aggr.at[pl.ds(s * rpt + i * K, K)])
    plsc.subcore_barrier()

    pltpu.sync_copy(sidx_h.at[pl.ds(wid * nchunk, nchunk)], sidx)
    pltpu.sync_copy(didx_h.at[pl.ds(wid * nchunk, nchunk)], didx)

    col_as = jnp.full((16,), D + D, jnp.int32)
    col_ad = jnp.full((16,), D, jnp.int32)

    def _issue(ci, b):
        pltpu.async_copy(stab.at[sidx.at[ci]], srows.at[b], semg.at[b])
        pltpu.async_copy(dtab.at[didx.at[ci]], drows.at[b], semg.at[b])

    def _wait_gather(ci, b):
        pltpu.make_async_copy(stab.at[sidx.at[ci]], srows.at[b],
                              semg.at[b]).wait()
        pltpu.make_async_copy(dtab.at[didx.at[ci]], drows.at[b],
                              semg.at[b]).wait()

    def _compute(ci, b):
        sr = srows.at[b]
        dr = drows.at[b]
        mg = msg.at[b]
        a_s = plsc.load_gather(sr, [iota, col_as])
        a_d = plsc.load_gather(dr, [iota, col_ad])
        logit = a_s + a_d
        ex = jnp.exp(jnp.maximum(logit, logit * 0.2))
        plsc.store_scatter(mg, [iota, col_ad], ex)

        @plsc.parallel_loop(0, K, 1, unroll=2)
        def _edge(j):
            jv = jnp.full((16,), j, jnp.int32)
            exv = plsc.load_gather(mg, [jv, col_ad])
            ts = [dr[j, pl.ds(f * 16, 16)] + sr[j, pl.ds(D + f * 16, 16)]
                  for f in range(D // 16)]
            es = [jnp.exp(-t) for t in ts]
            rs = [1.0 / (1.0 + e) for e in es]
            for f in range(D // 16):
                hj = sr[j, pl.ds(f * 16, 16)]
                mg[j, pl.ds(f * 16, 16)] = (exv * hj) * rs[f]
        pltpu.async_copy(msg.at[b], aggr.at[didx.at[ci]], sems.at[b],
                         add=True)

    def _wait_scatter(ci, b):
        pltpu.make_async_copy(msg.at[b], aggr.at[didx.at[ci]],
                              sems.at[b]).wait()

    _issue(0, 0)

    def _pair(m, _):
        c0 = 2 * m
        c1 = 2 * m + 1
        _wait_gather(c0, 0)
        _issue(c1, 1)

        @pl.when(m >= 1)
        def _():
            _wait_scatter(c0 - 2, 0)
        _compute(c0, 0)
        _wait_gather(c1, 1)
        if nchunk % 2:
            _issue(c1 + 1, 0)
        else:
            @pl.when(c1 + 1 < nchunk)
            def _():
                _issue(c1 + 1, 0)

        @pl.when(m >= 1)
        def _():
            _wait_scatter(c1 - 2, 1)
        _compute(c1, 1)
        return 0
    npairs = nchunk // 2
    lax.fori_loop(0, npairs, _pair, 0)

    if nchunk % 2:
        last = nchunk - 1
        _wait_gather(last, 0)
        _wait_scatter(last - 2, 0)
        _compute(last, 0)
        _wait_scatter(last - 1, 1)
        _wait_scatter(last, 0)
    else:
        _wait_scatter(nchunk - 2, 0)
        _wait_scatter(nchunk - 1, 1)

    plsc.subcore_barrier()
    full = n_nodes // rpt
    rem = n_nodes - full * rpt

    @pl.when(s < full)
    def _():
        pltpu.sync_copy(aggr.at[pl.ds(s * rpt, rpt)],
                        out.at[c, pl.ds(s * rpt, rpt)])
    if rem:
        @pl.when(s == full)
        def _():
            pltpu.sync_copy(aggr.at[pl.ds(full * rpt, rem)],
                            out.at[c, pl.ds(full * rpt, rem)])


def _sc_edge(src_tab, dst_tab, src_idx, dst_idx):
    n_nodes = src_tab.shape[0]
    n_edges = src_idx.shape[0]
    npad = ((n_nodes + NS * K - 1) // (NS * K)) * (NS * K)
    nchunk = n_edges // (NC * NS * K)
    mesh = plsc.VectorSubcoreMesh(core_axis_name="c", subcore_axis_name="s")
    run = pl.kernel(
        functools.partial(_sc_edge_body, n_nodes, n_edges),
        out_type=jax.ShapeDtypeStruct((NC, n_nodes, AGGW), F32),
        mesh=mesh,
        compiler_params=pltpu.CompilerParams(use_tc_tiling_on_sc=False,
                                             needs_layout_passes=False),
        scratch_types=[
            pltpu.VMEM((nchunk, 16), jnp.int32),
            pltpu.VMEM((nchunk, 16), jnp.int32),
            pltpu.VMEM((2, K, SRCW), F32),
            pltpu.VMEM((2, K, DSTW), F32),
            pltpu.VMEM((2, K, AGGW), F32),
            pltpu.SemaphoreType.DMA((2,)),
            pltpu.SemaphoreType.DMA((2,)),
            pltpu.VMEM_SHARED((npad, AGGW), F32),
        ],
    )
    return run(src_tab, dst_tab,
               src_idx.reshape(-1, 16), dst_idx.reshape(-1, 16))


def kernel(x, edge_index, params):
    n = x.shape[0]
    grid = (n // TC_ROWS,)
    src_idx = edge_index[0]
    dst_idx = edge_index[1]
    q0 = _prep_params(params["l0"])
    q1 = _prep_params(params["l1"])

    tab_shapes = [jax.ShapeDtypeStruct((n, SRCW), F32),
                  jax.ShapeDtypeStruct((n, DSTW), F32)]
    tab_specs = [_row_spec(SRCW), _row_spec(DSTW)]

    w0, w0_specs = _weight_args(q0)
    stab0, dtab0 = pl.pallas_call(
        _tables_body,
        grid=grid,
        in_specs=[_row_spec(D)] + w0_specs,
        out_specs=tab_specs,
        out_shape=tab_shapes,
    )(x, *w0)

    part0 = _sc_edge(stab0, dtab0, src_idx, dst_idx)

    w1, w1_specs = _weight_args(q1)
    stab1, dtab1 = pl.pallas_call(
        _combine_tables_body,
        grid=grid,
        in_specs=[pl.BlockSpec((NC, TC_ROWS, AGGW), lambda i: (0, i, 0)),
                  _row_spec(D), _full_spec((1, D)), _full_spec((1, D))]
                 + w1_specs,
        out_specs=tab_specs,
        out_shape=tab_shapes,
    )(part0, x, q0["g"], q0["b"], *w1)

    part1 = _sc_edge(stab1, dtab1, src_idx, dst_idx)

    out = pl.pallas_call(
        _combine_final_body,
        grid=grid,
        in_specs=[pl.BlockSpec((NC, TC_ROWS, AGGW), lambda i: (0, i, 0)),
                  _full_spec((1, D)), _full_spec((1, D))],
        out_specs=_row_spec(D),
        out_shape=jax.ShapeDtypeStruct((n, D), F32),
    )(part1, q1["g"], q1["b"])
    return out


# speedup vs baseline: 14.6293x; 1.6266x over previous
import functools

import jax
import jax.numpy as jnp
from jax import lax
from jax.experimental import pallas as pl
from jax.experimental.pallas import tpu as pltpu
from jax.experimental.pallas import tpu_sc as plsc

F32 = jnp.float32
D = 128
SRCW = 272
DSTW = 144
AGGW = 144
NC, NS = 2, 16
K = 16
NB = 3


def _idx_group(nchunk):
    return 125 if nchunk % 125 == 0 else nchunk
TC_ROWS = 1000


def _mm_t(x, w):
    return lax.dot_general(x, w, (((1,), (1,)), ((), ())),
                           preferred_element_type=F32)


def _build_tables(x, p, src_ref, dst_ref):
    h = _mm_t(x, p["W"][...]) + p["Wb"][...]
    hU = _mm_t(h, p["U"][...]) + p["Ub"][...]
    hV = _mm_t(h, p["V"][...]) + p["Vb"][...]
    a_s = jnp.dot(h, p["A2p"][...], preferred_element_type=F32)
    a_d = jnp.dot(h, p["A1p"][...], preferred_element_type=F32) + p["Abp"][...]
    src_ref[...] = jnp.concatenate([h, hV, a_s], axis=1)
    dst_ref[...] = jnp.concatenate([hU, a_d], axis=1)


def _tables_body(x_ref, W, Wb, U, Ub, V, Vb, A1p, A2p, Abp, src_ref, dst_ref):
    p = {"W": W, "Wb": Wb, "U": U, "Ub": Ub, "V": V, "Vb": Vb,
         "A1p": A1p, "A2p": A2p, "Abp": Abp}
    _build_tables(x_ref[...], p, src_ref, dst_ref)


def _combine(p_ref, g, b):
    psum = p_ref[0] + p_ref[1]
    s = psum[:, 128:129]
    aggr = psum[:, :D] / (s + 1e-16)
    mu = jnp.mean(aggr, axis=-1, keepdims=True)
    var = jnp.mean((aggr - mu) ** 2, axis=-1, keepdims=True)
    return (aggr - mu) * lax.rsqrt(var + 1e-5) * g[...] + b[...]


def _combine_tables_body(p_ref, x_ref, g0, b0,
                         W, Wb, U, Ub, V, Vb, A1p, A2p, Abp,
                         src_ref, dst_ref):
    y = _combine(p_ref, g0, b0)
    x1 = jnp.maximum(y + x_ref[...], 0.0)
    p = {"W": W, "Wb": Wb, "U": U, "Ub": Ub, "V": V, "Vb": Vb,
         "A1p": A1p, "A2p": A2p, "Abp": Abp}
    _build_tables(x1, p, src_ref, dst_ref)


def _combine_final_body(p_ref, g1, b1, out_ref):
    out_ref[...] = _combine(p_ref, g1, b1)


def _row_spec(w):
    return pl.BlockSpec((TC_ROWS, w), lambda i: (i, 0))


def _full_spec(shape):
    nd = len(shape)
    return pl.BlockSpec(shape, lambda i, _n=nd: (0,) * _n)


def _prep_params(p):
    A = p["A"]
    A1 = A[0, :D]
    A2 = A[0, D:]
    A1p = jnp.zeros((D, 16), F32).at[:, 0].set(A1)
    A2p = jnp.zeros((D, 16), F32).at[:, 0].set(A2)
    Abp = jnp.zeros((1, 16), F32).at[0, 0].set(p["Ab"][0])
    return {"W": p["W"], "Wb": p["Wb"].reshape(1, D),
            "U": p["U"], "Ub": p["Ub"].reshape(1, D),
            "V": p["V"], "Vb": p["Vb"].reshape(1, D),
            "A1p": A1p, "A2p": A2p, "Abp": Abp,
            "g": p["ln_g"].reshape(1, D), "b": p["ln_b"].reshape(1, D)}


def _weight_args(q):
    ws = [q["W"], q["Wb"], q["U"], q["Ub"], q["V"], q["Vb"],
          q["A1p"], q["A2p"], q["Abp"]]
    return ws, [_full_spec(w.shape) for w in ws]


def _sc_edge_body(n_nodes, n_edges,
                  stab, dtab, sidx_h, didx_h, out,
                  sidx, didx, srows, drows, msg, semi, semg, sems, aggr):
    ept = n_edges // (NC * NS)
    npad = ((n_nodes + NS * K - 1) // (NS * K)) * (NS * K)
    rpt = npad // NS
    nchunk = ept // K
    c = lax.axis_index("c")
    s = lax.axis_index("s")
    wid = c * NS + s

    zeros16 = jnp.zeros((16,), F32)
    iota = lax.iota(jnp.int32, 16)

    for m in range(NB):
        def _zm(j, _, _m=m):
            for gcol in range(AGGW // 16):
                msg[_m, j, pl.ds(gcol * 16, 16)] = zeros16
            return 0
        lax.fori_loop(0, K, _zm, 0)
    for i in range(rpt // K):
        pltpu.sync_copy(msg.at[0], aggr.at[pl.ds(s * rpt + i * K, K)])
    plsc.subcore_barrier()

    G = _idx_group(nchunk)
    ngroups = nchunk // G
    ebase = wid * nchunk

    def _idx_load(q, sync=False):
        gs = q % 2
        if sync:
            pltpu.sync_copy(sidx_h.at[pl.ds(ebase + q * G, G)], sidx.at[gs])
            pltpu.sync_copy(didx_h.at[pl.ds(ebase + q * G, G)], didx.at[gs])
        else:
            pltpu.async_copy(sidx_h.at[pl.ds(ebase + q * G, G)],
                             sidx.at[gs], semi.at[gs])
            pltpu.async_copy(didx_h.at[pl.ds(ebase + q * G, G)],
                             didx.at[gs], semi.at[gs])

    def _idx_wait(q):
        gs = q % 2
        pltpu.make_async_copy(sidx_h.at[pl.ds(ebase + q * G, G)],
                              sidx.at[gs], semi.at[gs]).wait()
        pltpu.make_async_copy(didx_h.at[pl.ds(ebase + q * G, G)],
                              didx.at[gs], semi.at[gs]).wait()

    def _srow(idx, ci):
        return idx.at[(ci // G) % 2, ci % G]

    col_as = jnp.full((16,), D + D, jnp.int32)
    col_ad = jnp.full((16,), D, jnp.int32)

    def _issue(ci, b):
        pltpu.async_copy(stab.at[_srow(sidx, ci)], srows.at[b], semg.at[b])
        pltpu.async_copy(dtab.at[_srow(didx, ci)], drows.at[b], semg.at[b])

    def _wait_gather(ci, b):
        pltpu.make_async_copy(stab.at[_srow(sidx, ci)], srows.at[b],
                              semg.at[b]).wait()
        pltpu.make_async_copy(dtab.at[_srow(didx, ci)], drows.at[b],
                              semg.at[b]).wait()

    def _compute(ci, b):
        sr = srows.at[b]
        dr = drows.at[b]
        mg = msg.at[b]
        a_s = plsc.load_gather(sr, [iota, col_as])
        a_d = plsc.load_gather(dr, [iota, col_ad])
        logit = a_s + a_d
        ex = jnp.exp(jnp.maximum(logit, logit * 0.2))
        plsc.store_scatter(mg, [iota, col_ad], ex)

        @plsc.parallel_loop(0, K, 1, unroll=2)
        def _edge(j):
            jv = jnp.full((16,), j, jnp.int32)
            exv = plsc.load_gather(mg, [jv, col_ad])
            ts = [dr[j, pl.ds(f * 16, 16)] + sr[j, pl.ds(D + f * 16, 16)]
                  for f in range(D // 16)]
            es = [jnp.exp(-t) for t in ts]
            rs = [1.0 / (1.0 + e) for e in es]
            for f in range(D // 16):
                hj = sr[j, pl.ds(f * 16, 16)]
                mg[j, pl.ds(f * 16, 16)] = (exv * hj) * rs[f]
        pltpu.async_copy(msg.at[b], aggr.at[_srow(didx, ci)], sems.at[b],
                         add=True)

    def _wait_scatter(ci, b):
        pltpu.make_async_copy(msg.at[b], aggr.at[_srow(didx, ci)],
                              sems.at[b]).wait()

    _idx_load(0, sync=True)
    if ngroups > 1:
        _idx_load(1)
    for p in range(NB - 1):
        _issue(p, p)

    def _step(ci, _):
        b = ci % NB
        nxt = ci + (NB - 1)
        _wait_gather(ci, b)

        @pl.when((nxt < nchunk) & (lax.rem(nxt, G) == 0))
        def _():
            q = nxt // G
            _idx_wait(q)

            @pl.when(q + 1 < ngroups)
            def _():
                _idx_load(q + 1)

        @pl.when(nxt < nchunk)
        def _():
            _issue(nxt, nxt % NB)

        @pl.when(ci >= NB)
        def _():
            _wait_scatter(ci - NB, b)
        _compute(ci, b)
        return 0
    lax.fori_loop(0, nchunk, _step, 0)
    for t in range(NB):
        ci = nchunk - NB + t
        _wait_scatter(ci, ci % NB)

    plsc.subcore_barrier()
    full = n_nodes // rpt
    rem = n_nodes - full * rpt

    @pl.when(s < full)
    def _():
        pltpu.sync_copy(aggr.at[pl.ds(s * rpt, rpt)],
                        out.at[c, pl.ds(s * rpt, rpt)])
    if rem:
        @pl.when(s == full)
        def _():
            pltpu.sync_copy(aggr.at[pl.ds(full * rpt, rem)],
                            out.at[c, pl.ds(full * rpt, rem)])


def _sc_edge(src_tab, dst_tab, src_idx, dst_idx):
    n_nodes = src_tab.shape[0]
    n_edges = src_idx.shape[0]
    npad = ((n_nodes + NS * K - 1) // (NS * K)) * (NS * K)
    nchunk = n_edges // (NC * NS * K)
    mesh = plsc.VectorSubcoreMesh(core_axis_name="c", subcore_axis_name="s")
    run = pl.kernel(
        functools.partial(_sc_edge_body, n_nodes, n_edges),
        out_type=jax.ShapeDtypeStruct((NC, n_nodes, AGGW), F32),
        mesh=mesh,
        compiler_params=pltpu.CompilerParams(use_tc_tiling_on_sc=False,
                                             needs_layout_passes=False),
        scratch_types=[
            pltpu.VMEM((2, _idx_group(nchunk), 16), jnp.int32),
            pltpu.VMEM((2, _idx_group(nchunk), 16), jnp.int32),
            pltpu.VMEM((NB, K, SRCW), F32),
            pltpu.VMEM((NB, K, DSTW), F32),
            pltpu.VMEM((NB, K, AGGW), F32),
            pltpu.SemaphoreType.DMA((2,)),
            pltpu.SemaphoreType.DMA((NB,)),
            pltpu.SemaphoreType.DMA((NB,)),
            pltpu.VMEM_SHARED((npad, AGGW), F32),
        ],
    )
    return run(src_tab, dst_tab,
               src_idx.reshape(-1, 16), dst_idx.reshape(-1, 16))


def kernel(x, edge_index, params):
    n = x.shape[0]
    grid = (n // TC_ROWS,)
    src_idx = edge_index[0]
    dst_idx = edge_index[1]
    q0 = _prep_params(params["l0"])
    q1 = _prep_params(params["l1"])

    tab_shapes = [jax.ShapeDtypeStruct((n, SRCW), F32),
                  jax.ShapeDtypeStruct((n, DSTW), F32)]
    tab_specs = [_row_spec(SRCW), _row_spec(DSTW)]

    w0, w0_specs = _weight_args(q0)
    stab0, dtab0 = pl.pallas_call(
        _tables_body,
        grid=grid,
        in_specs=[_row_spec(D)] + w0_specs,
        out_specs=tab_specs,
        out_shape=tab_shapes,
    )(x, *w0)

    part0 = _sc_edge(stab0, dtab0, src_idx, dst_idx)

    w1, w1_specs = _weight_args(q1)
    stab1, dtab1 = pl.pallas_call(
        _combine_tables_body,
        grid=grid,
        in_specs=[pl.BlockSpec((NC, TC_ROWS, AGGW), lambda i: (0, i, 0)),
                  _row_spec(D), _full_spec((1, D)), _full_spec((1, D))]
                 + w1_specs,
        out_specs=tab_specs,
        out_shape=tab_shapes,
    )(part0, x, q0["g"], q0["b"], *w1)

    part1 = _sc_edge(stab1, dtab1, src_idx, dst_idx)

    out = pl.pallas_call(
        _combine_final_body,
        grid=grid,
        in_specs=[pl.BlockSpec((NC, TC_ROWS, AGGW), lambda i: (0, i, 0)),
                  _full_spec((1, D)), _full_spec((1, D))],
        out_specs=_row_spec(D),
        out_shape=jax.ShapeDtypeStruct((n, D), F32),
    )(part1, q1["g"], q1["b"])
    return out
